# baseline (speedup 1.0000x reference)
import jax
import jax.numpy as jnp
from jax.experimental import pallas as pl
from jax.experimental.pallas import tpu as pltpu

_B = 16
_S = 48
_N = _B * _S
_E = 3072
_H = 128
_NH = 8
_DH = _H // _NH
_L = 6
_K = 16
_EG = _E // _B
_F32 = jnp.float32


def _ln(x):
    m = x.mean(-1, keepdims=True)
    v = ((x - m) ** 2).mean(-1, keepdims=True)
    return (x - m) / jnp.sqrt(v + 1e-5)


def _body(af_ref, mass_ref, pos_ref, bf_ref, pidx_ref,
          atomtab_ref, massc_ref, massW_ref, bondtab_ref, distc_ref, distW_ref,
          wq_ref, wk_ref, wv_ref, wo_ref, wb_ref, we_ref, out_ref):
    af = af_ref[0]
    x = jnp.zeros((_S, _H), dtype=_F32)
    iota_n = jax.lax.broadcasted_iota(jnp.int32, (_S, 32), 1)
    for f in range(9):
        ohf = (af[:, f:f + 1] == iota_n).astype(_F32)
        x = x + jnp.dot(ohf, atomtab_ref[f], preferred_element_type=_F32)
    mass = mass_ref[0]
    rbf_m = jnp.exp(-10.0 * (mass - massc_ref[:]) ** 2)
    x = x + jnp.dot(rbf_m, massW_ref[:], preferred_element_type=_F32)

    pos = pos_ref[0]
    p_i = jnp.broadcast_to(pos[:, None, :], (_S, _S, 3)).reshape(_S * _S, 3)
    p_j = jnp.broadcast_to(pos[None, :, :], (_S, _S, 3)).reshape(_S * _S, 3)
    d2 = ((p_i - p_j) ** 2).sum(axis=-1, keepdims=True)
    dist = jnp.sqrt(d2 + 1e-9)
    rbf_d = jnp.exp(-10.0 * (dist - distc_ref[:]) ** 2)
    e = jnp.dot(rbf_d, distW_ref[:], preferred_element_type=_F32)

    bf = bf_ref[0]
    iota_b = jax.lax.broadcasted_iota(jnp.int32, (_EG, 8), 1)
    e_emb = jnp.zeros((_EG, _H), dtype=_F32)
    for f in range(3):
        ohf = (bf[:, f:f + 1] == iota_b).astype(_F32)
        e_emb = e_emb + jnp.dot(ohf, bondtab_ref[f], preferred_element_type=_F32)
    pidx = pidx_ref[0]
    iota_p = jax.lax.broadcasted_iota(jnp.int32, (_EG, _S * _S), 1)
    ohs = (pidx == iota_p).astype(_F32)
    e = e + jax.lax.dot_general(ohs, e_emb, (((0,), (0,)), ((), ())),
                                preferred_element_type=_F32)

    maskh = (jax.lax.broadcasted_iota(jnp.int32, (_NH, _H), 1) // _DH
             == jax.lax.broadcasted_iota(jnp.int32, (_NH, _H), 0)).astype(_F32)

    scale = 1.0 / (float(_DH) ** 0.5)
    maskh_s = maskh * scale
    for l in range(_L):
        xn = _ln(x)
        q = jnp.dot(xn, wq_ref[l], preferred_element_type=_F32)
        k = jnp.dot(xn, wk_ref[l], preferred_element_type=_F32)
        v = jnp.dot(xn, wv_ref[l], preferred_element_type=_F32)
        q_sel = (q[:, None, :] * maskh_s).reshape(_S * _NH, _H)
        qk = jax.lax.dot_general(q_sel, k, (((1,), (1,)), ((), ())),
                                 preferred_element_type=_F32)
        eb = jnp.dot(e, wb_ref[l], preferred_element_type=_F32)
        eb_t = jnp.swapaxes(eb.reshape(_S, _S, _NH), 1, 2)
        lg3 = qk.reshape(_S, _NH, _S) + eb_t
        a3 = jnp.exp(lg3)
        s = jnp.sum(a3, axis=-1)
        av = jnp.dot(a3.reshape(_S * _NH, _S), v, preferred_element_type=_F32)
        out = (av.reshape(_S, _NH, _H) * maskh).sum(axis=1)
        den = jnp.dot(s, maskh, preferred_element_type=_F32)
        out = out / den
        x = x + jnp.dot(out, wo_ref[l], preferred_element_type=_F32)
        m = x[:, None, :] + x[None, :, :] + e.reshape(_S, _S, _H)
        relu_m = jnp.maximum(m, 0.0).reshape(_S * _S, _H)
        e = e + jnp.dot(relu_m, we_ref[l], preferred_element_type=_F32)

    out_ref[0] = x


def kernel(atom_feats, mass, bond_feats, pos, edge_index, node2graph, atom_tables,
           mass_centers, mass_W, bond_tables, dist_centers, dist_W, Wq, Wk, Wv, Wo, Wb, We):
    af = atom_feats.astype(jnp.int32).reshape(_B, _S, 9)
    mass3 = mass.reshape(_B, _S, 1)
    bf = bond_feats.astype(jnp.int32).reshape(_B, _EG, 3)
    pos3 = pos.reshape(_B, _S, 3)
    src = edge_index[0].astype(jnp.int32)
    dst = edge_index[1].astype(jnp.int32)
    pidx = ((src % _S) * _S + (dst % _S)).reshape(_B, _EG, 1)
    massc = mass_centers.reshape(1, _K).astype(_F32)
    distc = dist_centers.reshape(1, _K).astype(_F32)

    grid = (_B,)
    full = lambda shape: pl.BlockSpec(shape, lambda b: (0,) * len(shape))
    perg = lambda shape: pl.BlockSpec((1,) + shape, lambda b: (b,) + (0,) * len(shape))

    return pl.pallas_call(
        _body,
        grid=grid,
        in_specs=[
            perg((_S, 9)),
            perg((_S, 1)),
            perg((_S, 3)),
            perg((_EG, 3)),
            perg((_EG, 1)),
            full((9, 32, _H)),
            full((1, _K)),
            full((_K, _H)),
            full((3, 8, _H)),
            full((1, _K)),
            full((_K, _H)),
            full((_L, _H, _H)),
            full((_L, _H, _H)),
            full((_L, _H, _H)),
            full((_L, _H, _H)),
            full((_L, _H, _NH)),
            full((_L, _H, _H)),
        ],
        out_specs=pl.BlockSpec((1, _S, _H), lambda b: (b, 0, 0)),
        out_shape=jax.ShapeDtypeStruct((_B, _S, _H), _F32),
        compiler_params=pltpu.CompilerParams(dimension_semantics=("parallel",)),
    )(af, mass3, pos3, bf, pidx, atom_tables, massc, mass_W, bond_tables,
      distc, dist_W, Wq, Wk, Wv, Wo, Wb, We)


# speedup vs baseline: 1.0272x; 1.0272x over previous
import jax
import jax.numpy as jnp
from jax.experimental import pallas as pl
from jax.experimental.pallas import tpu as pltpu

_B = 16
_G = 2
_S = 48
_N = _B * _S
_E = 3072
_H = 128
_NH = 8
_DH = _H // _NH
_L = 6
_K = 16
_EG = _E // _B
_F32 = jnp.float32


def _ln(x):
    m = x.mean(-1, keepdims=True)
    v = ((x - m) ** 2).mean(-1, keepdims=True)
    return (x - m) / jnp.sqrt(v + 1e-5)


def _body(af_ref, mass_ref, pos_ref, bf_ref, pidx_ref,
          atomtab_ref, massc_ref, massW_ref, bondtab_ref, distc_ref, distW_ref,
          wq_ref, wk_ref, wv_ref, wo_ref, wb_ref, we_ref, out_ref):
    iota_n = jax.lax.broadcasted_iota(jnp.int32, (_S, 32), 1)
    iota_b = jax.lax.broadcasted_iota(jnp.int32, (_EG, 8), 1)
    iota_p = jax.lax.broadcasted_iota(jnp.int32, (_EG, _S * _S), 1)
    maskh = (jax.lax.broadcasted_iota(jnp.int32, (_NH, _H), 1) // _DH
             == jax.lax.broadcasted_iota(jnp.int32, (_NH, _H), 0)).astype(_F32)
    scale = 1.0 / (float(_DH) ** 0.5)
    maskh_s = maskh * scale

    xs = []
    es = []
    for g in range(_G):
        af = af_ref[g]
        x = jnp.zeros((_S, _H), dtype=_F32)
        for f in range(9):
            ohf = (af[:, f:f + 1] == iota_n).astype(_F32)
            x = x + jnp.dot(ohf, atomtab_ref[f], preferred_element_type=_F32)
        mass = mass_ref[g]
        rbf_m = jnp.exp(-10.0 * (mass - massc_ref[:]) ** 2)
        x = x + jnp.dot(rbf_m, massW_ref[:], preferred_element_type=_F32)

        pos = pos_ref[g]
        p_i = jnp.broadcast_to(pos[:, None, :], (_S, _S, 3)).reshape(_S * _S, 3)
        p_j = jnp.broadcast_to(pos[None, :, :], (_S, _S, 3)).reshape(_S * _S, 3)
        d2 = ((p_i - p_j) ** 2).sum(axis=-1, keepdims=True)
        dist = jnp.sqrt(d2 + 1e-9)
        rbf_d = jnp.exp(-10.0 * (dist - distc_ref[:]) ** 2)
        e = jnp.dot(rbf_d, distW_ref[:], preferred_element_type=_F32)

        bf = bf_ref[g]
        e_emb = jnp.zeros((_EG, _H), dtype=_F32)
        for f in range(3):
            ohf = (bf[:, f:f + 1] == iota_b).astype(_F32)
            e_emb = e_emb + jnp.dot(ohf, bondtab_ref[f], preferred_element_type=_F32)
        pidx = pidx_ref[g]
        ohs = (pidx == iota_p).astype(_F32)
        e = e + jax.lax.dot_general(ohs, e_emb, (((0,), (0,)), ((), ())),
                                    preferred_element_type=_F32)
        xs.append(x)
        es.append(e)

    for l in range(_L):
        for g in range(_G):
            x, e = xs[g], es[g]
            xn = _ln(x)
            q = jnp.dot(xn, wq_ref[l], preferred_element_type=_F32)
            k = jnp.dot(xn, wk_ref[l], preferred_element_type=_F32)
            v = jnp.dot(xn, wv_ref[l], preferred_element_type=_F32)
            q_sel = (q[:, None, :] * maskh_s).reshape(_S * _NH, _H)
            qk = jax.lax.dot_general(q_sel, k, (((1,), (1,)), ((), ())),
                                     preferred_element_type=_F32)
            eb = jnp.dot(e, wb_ref[l], preferred_element_type=_F32)
            eb_t = jnp.swapaxes(eb.reshape(_S, _S, _NH), 1, 2)
            lg3 = qk.reshape(_S, _NH, _S) + eb_t
            a3 = jnp.exp(lg3)
            s = jnp.sum(a3, axis=-1)
            av = jnp.dot(a3.reshape(_S * _NH, _S), v, preferred_element_type=_F32)
            out = (av.reshape(_S, _NH, _H) * maskh).sum(axis=1)
            den = jnp.dot(s, maskh, preferred_element_type=_F32)
            out = out / den
            x = x + jnp.dot(out, wo_ref[l], preferred_element_type=_F32)
            m = x[:, None, :] + x[None, :, :] + e.reshape(_S, _S, _H)
            relu_m = jnp.maximum(m, 0.0).reshape(_S * _S, _H)
            e = e + jnp.dot(relu_m, we_ref[l], preferred_element_type=_F32)
            xs[g], es[g] = x, e

    for g in range(_G):
        out_ref[g] = xs[g]


def kernel(atom_feats, mass, bond_feats, pos, edge_index, node2graph, atom_tables,
           mass_centers, mass_W, bond_tables, dist_centers, dist_W, Wq, Wk, Wv, Wo, Wb, We):
    af = atom_feats.astype(jnp.int32).reshape(_B, _S, 9)
    mass3 = mass.reshape(_B, _S, 1)
    bf = bond_feats.astype(jnp.int32).reshape(_B, _EG, 3)
    pos3 = pos.reshape(_B, _S, 3)
    src = edge_index[0].astype(jnp.int32)
    dst = edge_index[1].astype(jnp.int32)
    pidx = ((src % _S) * _S + (dst % _S)).reshape(_B, _EG, 1)
    massc = mass_centers.reshape(1, _K).astype(_F32)
    distc = dist_centers.reshape(1, _K).astype(_F32)

    grid = (_B // _G,)
    full = lambda shape: pl.BlockSpec(shape, lambda b: (0,) * len(shape))
    perg = lambda shape: pl.BlockSpec((_G,) + shape, lambda b: (b,) + (0,) * len(shape))

    return pl.pallas_call(
        _body,
        grid=grid,
        in_specs=[
            perg((_S, 9)),
            perg((_S, 1)),
            perg((_S, 3)),
            perg((_EG, 3)),
            perg((_EG, 1)),
            full((9, 32, _H)),
            full((1, _K)),
            full((_K, _H)),
            full((3, 8, _H)),
            full((1, _K)),
            full((_K, _H)),
            full((_L, _H, _H)),
            full((_L, _H, _H)),
            full((_L, _H, _H)),
            full((_L, _H, _H)),
            full((_L, _H, _NH)),
            full((_L, _H, _H)),
        ],
        out_specs=pl.BlockSpec((_G, _S, _H), lambda b: (b, 0, 0)),
        out_shape=jax.ShapeDtypeStruct((_B, _S, _H), _F32),
        compiler_params=pltpu.CompilerParams(dimension_semantics=("parallel",)),
    )(af, mass3, pos3, bf, pidx, atom_tables, massc, mass_W, bond_tables,
      distc, dist_W, Wq, Wk, Wv, Wo, Wb, We)


# speedup vs baseline: 1.0927x; 1.0638x over previous
import jax
import jax.numpy as jnp
from jax.experimental import pallas as pl
from jax.experimental.pallas import tpu as pltpu

_B = 16
_G = 2
_S = 48
_GS = _G * _S
_N = _B * _S
_E = 3072
_H = 128
_NH = 8
_DH = _H // _NH
_L = 6
_K = 16
_EG = _E // _B
_F32 = jnp.float32


def _ln(x):
    m = x.mean(-1, keepdims=True)
    v = ((x - m) ** 2).mean(-1, keepdims=True)
    return (x - m) / jnp.sqrt(v + 1e-5)


def _body(af_ref, mass_ref, pos_ref, bf_ref, pidx_ref,
          atomtab_ref, massc_ref, massW_ref, bondtab_ref, distc_ref, distW_ref,
          wq_ref, wk_ref, wv_ref, wo_ref, wb_ref, we_ref, out_ref):
    af = af_ref[0]
    iota_n = jax.lax.broadcasted_iota(jnp.int32, (_GS, 32), 1)
    x = jnp.zeros((_GS, _H), dtype=_F32)
    for f in range(9):
        ohf = (af[:, f:f + 1] == iota_n).astype(_F32)
        x = x + jnp.dot(ohf, atomtab_ref[f], preferred_element_type=_F32)
    mass = mass_ref[0]
    rbf_m = jnp.exp(-10.0 * (mass - massc_ref[:]) ** 2)
    x = x + jnp.dot(rbf_m, massW_ref[:], preferred_element_type=_F32)

    pos = pos_ref[0]
    p_i = jnp.broadcast_to(pos[:, None, :], (_GS, _S, 3)).reshape(_GS * _S, 3)
    p_j = jnp.broadcast_to(pos.reshape(_G, 1, _S, 3),
                           (_G, _S, _S, 3)).reshape(_GS * _S, 3)
    d2 = ((p_i - p_j) ** 2).sum(axis=-1, keepdims=True)
    dist = jnp.sqrt(d2 + 1e-9)
    rbf_d = jnp.exp(-10.0 * (dist - distc_ref[:]) ** 2)
    e = jnp.dot(rbf_d, distW_ref[:], preferred_element_type=_F32)

    bf = bf_ref[0].reshape(_G * _EG, 3)
    iota_b = jax.lax.broadcasted_iota(jnp.int32, (_G * _EG, 8), 1)
    e_emb = jnp.zeros((_G * _EG, _H), dtype=_F32)
    for f in range(3):
        ohf = (bf[:, f:f + 1] == iota_b).astype(_F32)
        e_emb = e_emb + jnp.dot(ohf, bondtab_ref[f], preferred_element_type=_F32)
    iota_p = jax.lax.broadcasted_iota(jnp.int32, (_EG, _S * _S), 1)
    deltas = []
    for g in range(_G):
        ohs = (pidx_ref[0, g] == iota_p).astype(_F32)
        deltas.append(jax.lax.dot_general(
            ohs, e_emb[g * _EG:(g + 1) * _EG], (((0,), (0,)), ((), ())),
            preferred_element_type=_F32))
    e = e + jnp.concatenate(deltas, axis=0)

    maskh = (jax.lax.broadcasted_iota(jnp.int32, (_NH, _H), 1) // _DH
             == jax.lax.broadcasted_iota(jnp.int32, (_NH, _H), 0)).astype(_F32)
    maskh_s = maskh * (1.0 / (float(_DH) ** 0.5))
    xg_row = jax.lax.broadcasted_iota(jnp.int32, (_GS, _NH, _GS), 0) // _S
    xg_col = jax.lax.broadcasted_iota(jnp.int32, (_GS, _NH, _GS), 2) // _S
    neg = jnp.where(xg_row == xg_col, 0.0, -1e30).astype(_F32)

    zcols = jnp.zeros((_S, _NH, _S), dtype=_F32)
    for l in range(_L):
        xn = _ln(x)
        q = jnp.dot(xn, wq_ref[l], preferred_element_type=_F32)
        k = jnp.dot(xn, wk_ref[l], preferred_element_type=_F32)
        v = jnp.dot(xn, wv_ref[l], preferred_element_type=_F32)
        q_sel = (q[:, None, :] * maskh_s).reshape(_GS * _NH, _H)
        qk = jax.lax.dot_general(q_sel, k, (((1,), (1,)), ((), ())),
                                 preferred_element_type=_F32)
        eb = jnp.dot(e, wb_ref[l], preferred_element_type=_F32)
        ebt = jnp.swapaxes(eb.reshape(_GS, _S, _NH), 1, 2)
        ebt_full = jnp.concatenate(
            [jnp.concatenate([ebt[:_S], zcols], axis=2),
             jnp.concatenate([zcols, ebt[_S:]], axis=2)], axis=0)
        lg3 = qk.reshape(_GS, _NH, _GS) + ebt_full + neg
        a3 = jnp.exp(lg3)
        s = jnp.sum(a3, axis=-1)
        av = jnp.dot(a3.reshape(_GS * _NH, _GS), v, preferred_element_type=_F32)
        out = (av.reshape(_GS, _NH, _H) * maskh).sum(axis=1)
        den = jnp.dot(s, maskh, preferred_element_type=_F32)
        out = out / den
        x = x + jnp.dot(out, wo_ref[l], preferred_element_type=_F32)
        xi = x[:, None, :]
        xj = jnp.broadcast_to(x.reshape(_G, 1, _S, _H),
                              (_G, _S, _S, _H)).reshape(_GS, _S, _H)
        m = xi + xj + e.reshape(_GS, _S, _H)
        relu_m = jnp.maximum(m, 0.0).reshape(_GS * _S, _H)
        e = e + jnp.dot(relu_m, we_ref[l], preferred_element_type=_F32)

    out_ref[0] = x


def kernel(atom_feats, mass, bond_feats, pos, edge_index, node2graph, atom_tables,
           mass_centers, mass_W, bond_tables, dist_centers, dist_W, Wq, Wk, Wv, Wo, Wb, We):
    af = atom_feats.astype(jnp.int32).reshape(_B // _G, _GS, 9)
    mass3 = mass.reshape(_B // _G, _GS, 1)
    bf = bond_feats.astype(jnp.int32).reshape(_B // _G, _G, _EG, 3)
    pos3 = pos.reshape(_B // _G, _GS, 3)
    src = edge_index[0].astype(jnp.int32)
    dst = edge_index[1].astype(jnp.int32)
    pidx = ((src % _S) * _S + (dst % _S)).reshape(_B // _G, _G, _EG, 1)
    massc = mass_centers.reshape(1, _K).astype(_F32)
    distc = dist_centers.reshape(1, _K).astype(_F32)

    grid = (_B // _G,)
    full = lambda shape: pl.BlockSpec(shape, lambda b: (0,) * len(shape))
    perg = lambda shape: pl.BlockSpec((1,) + shape, lambda b: (b,) + (0,) * len(shape))

    out = pl.pallas_call(
        _body,
        grid=grid,
        in_specs=[
            perg((_GS, 9)),
            perg((_GS, 1)),
            perg((_GS, 3)),
            perg((_G, _EG, 3)),
            perg((_G, _EG, 1)),
            full((9, 32, _H)),
            full((1, _K)),
            full((_K, _H)),
            full((3, 8, _H)),
            full((1, _K)),
            full((_K, _H)),
            full((_L, _H, _H)),
            full((_L, _H, _H)),
            full((_L, _H, _H)),
            full((_L, _H, _H)),
            full((_L, _H, _NH)),
            full((_L, _H, _H)),
        ],
        out_specs=pl.BlockSpec((1, _GS, _H), lambda b: (b, 0, 0)),
        out_shape=jax.ShapeDtypeStruct((_B // _G, _GS, _H), _F32),
        compiler_params=pltpu.CompilerParams(dimension_semantics=("parallel",)),
    )(af, mass3, pos3, bf, pidx, atom_tables, massc, mass_W, bond_tables,
      distc, dist_W, Wq, Wk, Wv, Wo, Wb, We)
    return out.reshape(_B, _S, _H)


# speedup vs baseline: 1.1998x; 1.0980x over previous
import jax
import jax.numpy as jnp
from jax.experimental import pallas as pl
from jax.experimental.pallas import tpu as pltpu

_B = 16
_G = 2
_S = 48
_GS = _G * _S
_N = _B * _S
_E = 3072
_H = 128
_NH = 8
_DH = _H // _NH
_L = 6
_K = 16
_EG = _E // _B
_F32 = jnp.float32


def _ln(x):
    m = x.mean(-1, keepdims=True)
    v = ((x - m) ** 2).mean(-1, keepdims=True)
    return (x - m) / jnp.sqrt(v + 1e-5)


def _body(af_ref, mass_ref, pos_ref, bf_ref, pidx_ref,
          atomtab_ref, massc_ref, massW_ref, bondtab_ref, distc_ref, distW_ref,
          wq_ref, wk_ref, wv_ref, wo_ref, wb_ref, we_ref, out_ref):
    af = af_ref[0]
    iota_n = jax.lax.broadcasted_iota(jnp.int32, (_GS, 32), 1)
    x = jnp.zeros((_GS, _H), dtype=_F32)
    for f in range(9):
        ohf = (af[:, f:f + 1] == iota_n).astype(_F32)
        x = x + jnp.dot(ohf, atomtab_ref[f], preferred_element_type=_F32)
    mass = mass_ref[0]
    rbf_m = jnp.exp(-10.0 * (mass - massc_ref[:]) ** 2)
    x = x + jnp.dot(rbf_m, massW_ref[:], preferred_element_type=_F32)

    pos = pos_ref[0]
    p_i = jnp.broadcast_to(pos[:, None, :], (_GS, _S, 3)).reshape(_GS * _S, 3)
    p_j = jnp.broadcast_to(pos.reshape(_G, 1, _S, 3),
                           (_G, _S, _S, 3)).reshape(_GS * _S, 3)
    d2 = ((p_i - p_j) ** 2).sum(axis=-1, keepdims=True)
    dist = jnp.sqrt(d2 + 1e-9)
    rbf_d = jnp.exp(-10.0 * (dist - distc_ref[:]) ** 2)
    e = jnp.dot(rbf_d, distW_ref[:], preferred_element_type=_F32)

    bf = bf_ref[0].reshape(_G * _EG, 3)
    iota_b = jax.lax.broadcasted_iota(jnp.int32, (_G * _EG, 8), 1)
    e_emb = jnp.zeros((_G * _EG, _H), dtype=_F32)
    for f in range(3):
        ohf = (bf[:, f:f + 1] == iota_b).astype(_F32)
        e_emb = e_emb + jnp.dot(ohf, bondtab_ref[f], preferred_element_type=_F32)
    iota_p = jax.lax.broadcasted_iota(jnp.int32, (_EG, _S * _S), 1)
    deltas = []
    for g in range(_G):
        ohs = (pidx_ref[0, g] == iota_p).astype(_F32)
        deltas.append(jax.lax.dot_general(
            ohs, e_emb[g * _EG:(g + 1) * _EG], (((0,), (0,)), ((), ())),
            preferred_element_type=_F32))
    e = e + jnp.concatenate(deltas, axis=0)

    maskh = (jax.lax.broadcasted_iota(jnp.int32, (_NH, _H), 1) // _DH
             == jax.lax.broadcasted_iota(jnp.int32, (_NH, _H), 0)).astype(_F32)
    maskh_s = maskh * (1.0 / (float(_DH) ** 0.5))

    rg = range(_G)
    xs = [x[g * _S:(g + 1) * _S] for g in rg]
    es = [e[g * _S * _S:(g + 1) * _S * _S] for g in rg]

    for l in range(_L):
        wq, wk, wv = wq_ref[l], wk_ref[l], wv_ref[l]
        wo, wb, we = wo_ref[l], wb_ref[l], we_ref[l]
        xn = [_ln(xs[g]) for g in rg]
        q = [jnp.dot(xn[g], wq, preferred_element_type=_F32) for g in rg]
        k = [jnp.dot(xn[g], wk, preferred_element_type=_F32) for g in rg]
        v = [jnp.dot(xn[g], wv, preferred_element_type=_F32) for g in rg]
        qs = [(q[g][:, None, :] * maskh_s).reshape(_S * _NH, _H) for g in rg]
        qk = [jax.lax.dot_general(qs[g], k[g], (((1,), (1,)), ((), ())),
                                  preferred_element_type=_F32) for g in rg]
        eb = [jnp.dot(es[g], wb, preferred_element_type=_F32) for g in rg]
        ebt = [jnp.swapaxes(eb[g].reshape(_S, _S, _NH), 1, 2) for g in rg]
        lg3 = [qk[g].reshape(_S, _NH, _S) + ebt[g] for g in rg]
        a3 = [jnp.exp(lg3[g]) for g in rg]
        s = [jnp.sum(a3[g], axis=-1) for g in rg]
        av = [jnp.dot(a3[g].reshape(_S * _NH, _S), v[g],
                      preferred_element_type=_F32) for g in rg]
        out = [(av[g].reshape(_S, _NH, _H) * maskh).sum(axis=1) for g in rg]
        den = [jnp.dot(s[g], maskh, preferred_element_type=_F32) for g in rg]
        od = [out[g] / den[g] for g in rg]
        xs = [xs[g] + jnp.dot(od[g], wo, preferred_element_type=_F32) for g in rg]
        m = [xs[g][:, None, :] + xs[g][None, :, :] + es[g].reshape(_S, _S, _H)
             for g in rg]
        rm = [jnp.maximum(m[g], 0.0).reshape(_S * _S, _H) for g in rg]
        es = [es[g] + jnp.dot(rm[g], we, preferred_element_type=_F32) for g in rg]

    out_ref[0] = jnp.concatenate(xs, axis=0)


def kernel(atom_feats, mass, bond_feats, pos, edge_index, node2graph, atom_tables,
           mass_centers, mass_W, bond_tables, dist_centers, dist_W, Wq, Wk, Wv, Wo, Wb, We):
    af = atom_feats.astype(jnp.int32).reshape(_B // _G, _GS, 9)
    mass3 = mass.reshape(_B // _G, _GS, 1)
    bf = bond_feats.astype(jnp.int32).reshape(_B // _G, _G, _EG, 3)
    pos3 = pos.reshape(_B // _G, _GS, 3)
    src = edge_index[0].astype(jnp.int32)
    dst = edge_index[1].astype(jnp.int32)
    pidx = ((src % _S) * _S + (dst % _S)).reshape(_B // _G, _G, _EG, 1)
    massc = mass_centers.reshape(1, _K).astype(_F32)
    distc = dist_centers.reshape(1, _K).astype(_F32)

    grid = (_B // _G,)
    full = lambda shape: pl.BlockSpec(shape, lambda b: (0,) * len(shape))
    perg = lambda shape: pl.BlockSpec((1,) + shape, lambda b: (b,) + (0,) * len(shape))

    out = pl.pallas_call(
        _body,
        grid=grid,
        in_specs=[
            perg((_GS, 9)),
            perg((_GS, 1)),
            perg((_GS, 3)),
            perg((_G, _EG, 3)),
            perg((_G, _EG, 1)),
            full((9, 32, _H)),
            full((1, _K)),
            full((_K, _H)),
            full((3, 8, _H)),
            full((1, _K)),
            full((_K, _H)),
            full((_L, _H, _H)),
            full((_L, _H, _H)),
            full((_L, _H, _H)),
            full((_L, _H, _H)),
            full((_L, _H, _NH)),
            full((_L, _H, _H)),
        ],
        out_specs=pl.BlockSpec((1, _GS, _H), lambda b: (b, 0, 0)),
        out_shape=jax.ShapeDtypeStruct((_B // _G, _GS, _H), _F32),
        compiler_params=pltpu.CompilerParams(dimension_semantics=("parallel",)),
    )(af, mass3, pos3, bf, pidx, atom_tables, massc, mass_W, bond_tables,
      distc, dist_W, Wq, Wk, Wv, Wo, Wb, We)
    return out.reshape(_B, _S, _H)


# speedup vs baseline: 1.8273x; 1.5230x over previous
import jax
import jax.numpy as jnp
from jax.experimental import pallas as pl
from jax.experimental.pallas import tpu as pltpu

_B = 16
_G = 4
_S = 48
_GS = _G * _S
_N = _B * _S
_E = 3072
_H = 128
_NH = 8
_DH = _H // _NH
_L = 6
_K = 16
_EG = _E // _B
_F32 = jnp.float32


def _ln(x):
    m = x.mean(-1, keepdims=True)
    v = ((x - m) ** 2).mean(-1, keepdims=True)
    return (x - m) / jnp.sqrt(v + 1e-5)


def _body(af_ref, mass_ref, pos_ref, bf_ref, pidx_ref,
          atomtab_ref, massc_ref, massW_ref, bondtab_ref, distc_ref, distW_ref,
          wq_ref, wk_ref, wv_ref, wo_ref, wb_ref, we_ref, out_ref):
    af = af_ref[0]
    iota_n = jax.lax.broadcasted_iota(jnp.int32, (_GS, 32), 1)
    x = jnp.zeros((_GS, _H), dtype=_F32)
    for f in range(9):
        ohf = (af[:, f:f + 1] == iota_n).astype(_F32)
        x = x + jnp.dot(ohf, atomtab_ref[f], preferred_element_type=_F32)
    mass = mass_ref[0]
    rbf_m = jnp.exp(-10.0 * (mass - massc_ref[:]) ** 2)
    x = x + jnp.dot(rbf_m, massW_ref[:], preferred_element_type=_F32)

    pos = pos_ref[0]
    p_i = jnp.broadcast_to(pos[:, None, :], (_GS, _S, 3)).reshape(_GS * _S, 3)
    p_j = jnp.broadcast_to(pos.reshape(_G, 1, _S, 3),
                           (_G, _S, _S, 3)).reshape(_GS * _S, 3)
    d2 = ((p_i - p_j) ** 2).sum(axis=-1, keepdims=True)
    dist = jnp.sqrt(d2 + 1e-9)
    rbf_d = jnp.exp(-10.0 * (dist - distc_ref[:]) ** 2)
    e = jnp.dot(rbf_d, distW_ref[:], preferred_element_type=_F32)

    bf = bf_ref[0].reshape(_G * _EG, 3)
    iota_b = jax.lax.broadcasted_iota(jnp.int32, (_G * _EG, 8), 1)
    e_emb = jnp.zeros((_G * _EG, _H), dtype=_F32)
    for f in range(3):
        ohf = (bf[:, f:f + 1] == iota_b).astype(_F32)
        e_emb = e_emb + jnp.dot(ohf, bondtab_ref[f], preferred_element_type=_F32)
    iota_p = jax.lax.broadcasted_iota(jnp.int32, (_EG, _S * _S), 1)
    deltas = []
    for g in range(_G):
        ohs = (pidx_ref[0, g] == iota_p).astype(_F32)
        deltas.append(jax.lax.dot_general(
            ohs, e_emb[g * _EG:(g + 1) * _EG], (((0,), (0,)), ((), ())),
            preferred_element_type=_F32))
    e = e + jnp.concatenate(deltas, axis=0)

    maskh = (jax.lax.broadcasted_iota(jnp.int32, (_NH, _H), 1) // _DH
             == jax.lax.broadcasted_iota(jnp.int32, (_NH, _H), 0)).astype(_F32)
    maskh_s = maskh * (1.0 / (float(_DH) ** 0.5))

    rg = range(_G)
    xs = [x[g * _S:(g + 1) * _S] for g in rg]
    es = [e[g * _S * _S:(g + 1) * _S * _S] for g in rg]

    for l in range(_L):
        wq, wk, wv = wq_ref[l], wk_ref[l], wv_ref[l]
        wo, wb, we = wo_ref[l], wb_ref[l], we_ref[l]
        xn = [_ln(xs[g]) for g in rg]
        q = [jnp.dot(xn[g], wq, preferred_element_type=_F32) for g in rg]
        k = [jnp.dot(xn[g], wk, preferred_element_type=_F32) for g in rg]
        v = [jnp.dot(xn[g], wv, preferred_element_type=_F32) for g in rg]
        qs = [(q[g][:, None, :] * maskh_s).reshape(_S * _NH, _H) for g in rg]
        qk = [jax.lax.dot_general(qs[g], k[g], (((1,), (1,)), ((), ())),
                                  preferred_element_type=_F32) for g in rg]
        eb = [jnp.dot(es[g], wb, preferred_element_type=_F32) for g in rg]
        ebt = [jnp.swapaxes(eb[g].reshape(_S, _S, _NH), 1, 2) for g in rg]
        lg3 = [qk[g].reshape(_S, _NH, _S) + ebt[g] for g in rg]
        a3 = [jnp.exp(lg3[g]) for g in rg]
        s = [jnp.sum(a3[g], axis=-1) for g in rg]
        av = [jnp.dot(a3[g].reshape(_S * _NH, _S), v[g],
                      preferred_element_type=_F32) for g in rg]
        out = [(av[g].reshape(_S, _NH, _H) * maskh).sum(axis=1) for g in rg]
        den = [jnp.dot(s[g], maskh, preferred_element_type=_F32) for g in rg]
        od = [out[g] / den[g] for g in rg]
        xs = [xs[g] + jnp.dot(od[g], wo, preferred_element_type=_F32) for g in rg]
        m = [xs[g][:, None, :] + xs[g][None, :, :] + es[g].reshape(_S, _S, _H)
             for g in rg]
        rm = [jnp.maximum(m[g], 0.0).reshape(_S * _S, _H) for g in rg]
        es = [es[g] + jnp.dot(rm[g], we, preferred_element_type=_F32) for g in rg]

    out_ref[0] = jnp.concatenate(xs, axis=0)


def kernel(atom_feats, mass, bond_feats, pos, edge_index, node2graph, atom_tables,
           mass_centers, mass_W, bond_tables, dist_centers, dist_W, Wq, Wk, Wv, Wo, Wb, We):
    af = atom_feats.astype(jnp.int32).reshape(_B // _G, _GS, 9)
    mass3 = mass.reshape(_B // _G, _GS, 1)
    bf = bond_feats.astype(jnp.int32).reshape(_B // _G, _G, _EG, 3)
    pos3 = pos.reshape(_B // _G, _GS, 3)
    src = edge_index[0].astype(jnp.int32)
    dst = edge_index[1].astype(jnp.int32)
    pidx = ((src % _S) * _S + (dst % _S)).reshape(_B // _G, _G, _EG, 1)
    massc = mass_centers.reshape(1, _K).astype(_F32)
    distc = dist_centers.reshape(1, _K).astype(_F32)

    grid = (_B // _G,)
    full = lambda shape: pl.BlockSpec(shape, lambda b: (0,) * len(shape))
    perg = lambda shape: pl.BlockSpec((1,) + shape, lambda b: (b,) + (0,) * len(shape))

    out = pl.pallas_call(
        _body,
        grid=grid,
        in_specs=[
            perg((_GS, 9)),
            perg((_GS, 1)),
            perg((_GS, 3)),
            perg((_G, _EG, 3)),
            perg((_G, _EG, 1)),
            full((9, 32, _H)),
            full((1, _K)),
            full((_K, _H)),
            full((3, 8, _H)),
            full((1, _K)),
            full((_K, _H)),
            full((_L, _H, _H)),
            full((_L, _H, _H)),
            full((_L, _H, _H)),
            full((_L, _H, _H)),
            full((_L, _H, _NH)),
            full((_L, _H, _H)),
        ],
        out_specs=pl.BlockSpec((1, _GS, _H), lambda b: (b, 0, 0)),
        out_shape=jax.ShapeDtypeStruct((_B // _G, _GS, _H), _F32),
        compiler_params=pltpu.CompilerParams(dimension_semantics=("parallel",)),
    )(af, mass3, pos3, bf, pidx, atom_tables, massc, mass_W, bond_tables,
      distc, dist_W, Wq, Wk, Wv, Wo, Wb, We)
    return out.reshape(_B, _S, _H)
